# baseline (speedup 1.0000x reference)
import functools

import jax
import jax.numpy as jnp
from jax import lax
from jax.experimental import pallas as pl
from jax.experimental.pallas import tpu as pltpu
from jax.experimental.pallas import tpu_sc as plsc

N = 10000
E = 320000
D = 128
ROW_BLK = 2000

NC = 2
NS = 16
NW = NC * NS
RPW = 320
NPAD = NW * RPW
FLUSH = 8192
BUFCAP = FLUSH + 128
SCAN = 3200
NSCAN = E // SCAN
COMP_CAP = 40 * FLUSH
CHUNK = 128
GCHUNK = 256
NBIN = 336
CAP2 = 65536
CAP2B = CAP2 + FLUSH
RANK1 = 1

_mesh = plsc.VectorSubcoreMesh(
    core_axis_name="c", subcore_axis_name="s", num_cores=NC, num_subcores=NS)
_sc_params = pltpu.CompilerParams(needs_layout_passes=False)


def _wid():
    return lax.axis_index("s") * NC + lax.axis_index("c")


@functools.partial(
    pl.kernel,
    out_type=(
        jax.ShapeDtypeStruct((NW * COMP_CAP,), jnp.int32),
        jax.ShapeDtypeStruct((NW * COMP_CAP,), jnp.int32),
        jax.ShapeDtypeStruct((NW * 16,), jnp.int32),
        jax.ShapeDtypeStruct((NW * CAP2B,), jnp.int32),
        jax.ShapeDtypeStruct((NW * NBIN,), jnp.int32),
    ),
    mesh=_mesh,
    compiler_params=_sc_params,
    scratch_types=[
        pltpu.VMEM((SCAN,), jnp.int32),
        pltpu.VMEM((SCAN,), jnp.int32),
        pltpu.VMEM((SCAN,), jnp.int32),
        pltpu.VMEM((SCAN,), jnp.int32),
        pltpu.VMEM((BUFCAP,), jnp.int32),
        pltpu.VMEM((BUFCAP,), jnp.int32),
        pltpu.VMEM((NBIN,), jnp.int32),
        pltpu.VMEM((NBIN,), jnp.int32),
        pltpu.VMEM((CAP2B,), jnp.int32),
        pltpu.VMEM((16,), jnp.int32),
        pltpu.SemaphoreType.DMA,
        pltpu.SemaphoreType.DMA,
    ],
)
def _partition_edges(src_hbm, dst_hbm, comp_src, comp_dst, counts,
                     sorted_src, rowptr_out,
                     sbufA, sbufB, dbufA, dbufB, csrc, cdst,
                     hist, rowptr, sorted_buf, cbuf, semA, semB):
    w = _wid()
    lo = w * RPW
    base_out = w * COMP_CAP
    urpw = jnp.uint32(RPW)
    iota16 = lax.broadcasted_iota(jnp.int32, (16,), 0)

    zeros16 = jnp.zeros((16,), jnp.int32)
    trash16 = jnp.full((16,), RPW, jnp.int32)

    def init_body(k, _):
        for u in range(8):
            csrc[pl.ds((k * 8 + u) * 16, 16)] = zeros16
            cdst[pl.ds((k * 8 + u) * 16, 16)] = trash16
        return _
    lax.fori_loop(0, BUFCAP // 128, init_body, None)
    for k in range(NBIN // 16):
        hist[pl.ds(k * 16, 16)] = zeros16

    def start_load(c, sbuf, dbuf, sem):
        o = pl.multiple_of(c * SCAN, 8)
        pltpu.async_copy(src_hbm.at[pl.ds(o, SCAN)], sbuf, sem)
        pltpu.async_copy(dst_hbm.at[pl.ds(o, SCAN)], dbuf, sem)

    def wait_load(sbuf, dbuf, sem):
        pltpu.make_async_copy(src_hbm.at[pl.ds(0, SCAN)], sbuf, sem).wait()
        pltpu.make_async_copy(dst_hbm.at[pl.ds(0, SCAN)], dbuf, sem).wait()

    def scan_buf(sbuf, dbuf, pos, off):
        def vec_body(v, carry):
            pos, off = carry
            for u in range(8):
                b = v * 128 + u * 16
                d = dbuf[pl.ds(b, 16)]
                s = sbuf[pl.ds(b, 16)]
                dl = d - lo
                m = dl.astype(jnp.uint32) < urpw
                cnt = jnp.sum(m.astype(jnp.int32))
                plsc.store_compressed(csrc.at[pl.ds(pos, 16)], s, mask=m)
                plsc.store_compressed(cdst.at[pl.ds(pos, 16)], dl, mask=m)
                pos = pos + cnt

            def do_flush(pos, off):
                o = pl.multiple_of(base_out + off, FLUSH)
                pltpu.sync_copy(csrc.at[pl.ds(0, FLUSH)],
                                comp_src.at[pl.ds(o, FLUSH)])
                pltpu.sync_copy(cdst.at[pl.ds(0, FLUSH)],
                                comp_dst.at[pl.ds(o, FLUSH)])
                for t in range(8):
                    csrc[pl.ds(t * 16, 16)] = csrc[pl.ds(FLUSH + t * 16, 16)]
                    cdst[pl.ds(t * 16, 16)] = cdst[pl.ds(FLUSH + t * 16, 16)]
                return pos - FLUSH, off + FLUSH

            return lax.cond(pos >= FLUSH, do_flush, lambda p, o: (p, o),
                            pos, off)
        return lax.fori_loop(0, SCAN // 128, vec_body, (pos, off))

    start_load(0, sbufA, dbufA, semA)

    def pair_body(c2, carry):
        pos, off = carry
        cB = c2 * 2 + 1
        wait_load(sbufA, dbufA, semA)
        start_load(cB, sbufB, dbufB, semB)
        pos, off = scan_buf(sbufA, dbufA, pos, off)
        wait_load(sbufB, dbufB, semB)

        @pl.when(cB + 1 < NSCAN)
        def _prefA():
            start_load(cB + 1, sbufA, dbufA, semA)
        pos, off = scan_buf(sbufB, dbufB, pos, off)
        return pos, off
    pos, off = lax.fori_loop(0, NSCAN // 2, pair_body, (0, 0))

    o = pl.multiple_of(base_out + off, FLUSH)
    pltpu.sync_copy(csrc.at[pl.ds(0, FLUSH)],
                    comp_src.at[pl.ds(o, FLUSH)])
    pltpu.sync_copy(cdst.at[pl.ds(0, FLUSH)],
                    comp_dst.at[pl.ds(o, FLUSH)])
    cnt = off + pos
    cbuf[pl.ds(0, 16)] = jnp.full((16,), cnt, jnp.int32)
    pltpu.sync_copy(cbuf, counts.at[pl.ds(pl.multiple_of(w * 16, 16), 16)])

    @pl.when(cnt <= CAP2)
    def _sort():
        nh = (cnt + (SCAN - 1)) // SCAN

        def hist_chunk(hc, _):
            o = pl.multiple_of(base_out + hc * SCAN, 8)
            pltpu.sync_copy(comp_dst.at[pl.ds(o, SCAN)], dbufA)
            e0 = hc * SCAN

            def hvec(v, _):
                d = dbufA[pl.ds(v * 16, 16)]
                valid = (iota16 + (e0 + v * 16)) < cnt
                d = jnp.where(valid, d, RPW)
                rank, lastm = plsc.scan_count(d, mask=valid)
                plsc.addupdate_scatter(hist, [d], rank + (1 - RANK1),
                                       mask=lastm)
                return _
            lax.fori_loop(0, SCAN // 16, hvec, None)
            return _
        lax.fori_loop(0, nh, hist_chunk, None)

        def pfx(k, carry):
            v = hist[pl.ds(k * 16, 16)]
            incl = plsc.cumsum(v)
            rowptr[pl.ds(k * 16, 16)] = incl - v + carry
            return carry + jnp.max(incl)
        lax.fori_loop(0, NBIN // 16, pfx, 0)
        pltpu.sync_copy(
            rowptr, rowptr_out.at[pl.ds(pl.multiple_of(w * NBIN, 8), NBIN)])
        for k in range(NBIN // 16):
            hist[pl.ds(k * 16, 16)] = rowptr[pl.ds(k * 16, 16)]

        def zinit(k, _):
            sorted_buf[pl.ds(k * 16, 16)] = zeros16
            return _
        lax.fori_loop(0, (cnt + GCHUNK + 15) // 16, zinit, None)

        def scat_chunk(hc, _):
            o = pl.multiple_of(base_out + hc * SCAN, 8)
            pltpu.sync_copy(comp_dst.at[pl.ds(o, SCAN)], dbufA)
            pltpu.sync_copy(comp_src.at[pl.ds(o, SCAN)], sbufA)
            e0 = hc * SCAN

            def svec(v, _):
                d = dbufA[pl.ds(v * 16, 16)]
                s = sbufA[pl.ds(v * 16, 16)]
                valid = (iota16 + (e0 + v * 16)) < cnt
                d = jnp.where(valid, d, RPW)
                rank, lastm = plsc.scan_count(d, mask=valid)
                base = plsc.load_gather(hist, [d])
                plsc.store_scatter(sorted_buf, [base + (rank - RANK1)], s,
                                   mask=valid)
                plsc.addupdate_scatter(hist, [d], rank + (1 - RANK1),
                                       mask=lastm)
                return _
            lax.fori_loop(0, SCAN // 16, svec, None)
            return _
        lax.fori_loop(0, nh, scat_chunk, None)

        def pad_body(k, _):
            sorted_buf[pl.ds(cnt + k * 16, 16)] = zeros16
            return _
        lax.fori_loop(0, GCHUNK // 16, pad_body, None)

        npiece = (cnt + GCHUNK + (FLUSH - 1)) // FLUSH

        def piece(p, _):
            src_o = pl.multiple_of(p * FLUSH, 8)
            dst_o = pl.multiple_of(w * CAP2B + p * FLUSH, 8)
            pltpu.sync_copy(sorted_buf.at[pl.ds(src_o, FLUSH)],
                            sorted_src.at[pl.ds(dst_o, FLUSH)])
            return _
        lax.fori_loop(0, npiece, piece, None)


@functools.partial(
    pl.kernel,
    out_type=jax.ShapeDtypeStruct((NPAD, D), jnp.float32),
    mesh=_mesh,
    compiler_params=_sc_params,
    scratch_types=[
        pltpu.VMEM((GCHUNK,), jnp.int32),
        pltpu.VMEM((GCHUNK,), jnp.int32),
        pltpu.VMEM((GCHUNK,), jnp.int32),
        pltpu.VMEM((GCHUNK,), jnp.int32),
        pltpu.VMEM((GCHUNK, D), jnp.float32),
        pltpu.VMEM((GCHUNK, D), jnp.float32),
        pltpu.VMEM((RPW + 4, D), jnp.float32),
        pltpu.VMEM((NBIN,), jnp.int32),
        pltpu.SMEM((NBIN,), jnp.int32),
        pltpu.VMEM((16,), jnp.int32),
        pltpu.SemaphoreType.DMA,
        pltpu.SemaphoreType.DMA,
    ],
)
def _segmax_sc(h_hbm, comp_src, comp_dst, counts, sorted_src, rowptr_hbm, out,
               idxA, idxB, dstA, dstB, rowsA, rowsB, acc, rpv, rps,
               cbuf, semA, semB):
    w = _wid()
    base_in = w * COMP_CAP
    base_s = w * CAP2B

    pltpu.sync_copy(counts.at[pl.ds(pl.multiple_of(w * 16, 16), 16)], cbuf)
    cnt = jnp.max(cbuf[pl.ds(0, 16)])
    nchunks = (cnt + (GCHUNK - 1)) // GCHUNK

    neg16 = jnp.full((16,), -jnp.inf, jnp.float32)
    iota16 = lax.broadcasted_iota(jnp.int32, (16,), 0)

    def init_body(k, _):
        for u in range(4):
            for j in range(8):
                acc[k * 4 + u, pl.ds(j * 16, 16)] = neg16
        return _
    lax.fori_loop(0, (RPW + 4) // 4, init_body, None)

    @pl.when(cnt <= CAP2)
    def _fast():
        pltpu.sync_copy(
            rowptr_hbm.at[pl.ds(pl.multiple_of(w * NBIN, 8), NBIN)], rpv)

        def rp_body(k, _):
            vec = rpv[pl.ds(k * 16, 16)]
            for e in range(16):
                rps[k * 16 + e] = jnp.max(jnp.where(iota16 == e, vec, 0))
            return _
        lax.fori_loop(0, NBIN // 16, rp_body, None)

        def load_chunk(c, idxb, rows, sem):
            o = pl.multiple_of(base_s + c * GCHUNK, 8)
            pltpu.sync_copy(sorted_src.at[pl.ds(o, GCHUNK)], idxb)
            for k in range(GCHUNK // CHUNK):
                pltpu.async_copy(h_hbm.at[idxb.at[pl.ds(k * CHUNK, CHUNK)]],
                                 rows.at[pl.ds(k * CHUNK, CHUNK)], sem)

        def compute_chunk(c, rows, r):
            e0 = c * GCHUNK
            e1 = e0 + GCHUNK

            def run_cond(carry):
                r, done = carry
                return jnp.logical_and(
                    jnp.logical_not(done),
                    jnp.logical_and(r < RPW, rps[r] < e1))

            def run_body(carry):
                r, done = carry
                s0 = rps[r]
                s1 = rps[r + 1]
                lo_e = jnp.maximum(s0, e0) - e0
                hi_e = jnp.minimum(s1, e1) - e0

                @pl.when(hi_e > lo_e)
                def _accum():
                    vals = [acc[r, pl.ds(j * 16, 16)] for j in range(8)]

                    def emax2(i, vals):
                        e = lo_e + i * 2
                        return [jnp.maximum(
                                    jnp.maximum(vals[j],
                                                rows[e, pl.ds(j * 16, 16)]),
                                    rows[e + 1, pl.ds(j * 16, 16)])
                                for j in range(8)]
                    n2 = (hi_e - lo_e) // 2
                    vals = lax.fori_loop(0, n2, emax2, vals)

                    def vtail(vals):
                        e = lo_e + n2 * 2
                        return [jnp.maximum(vals[j],
                                            rows[e, pl.ds(j * 16, 16)])
                                for j in range(8)]
                    vals = lax.cond((hi_e - lo_e) % 2 == 1, vtail,
                                    lambda v: v, vals)
                    for j in range(8):
                        acc[r, pl.ds(j * 16, 16)] = vals[j]

                adv = s1 <= e1
                return jnp.where(adv, r + 1, r), jnp.logical_not(adv)

            r, _done = lax.while_loop(run_cond, run_body, (r, False))
            return r

        @pl.when(nchunks > 0)
        def _prologue():
            load_chunk(0, idxA, rowsA, semA)

        def pair_body(c2, r):
            cA = c2 * 2
            cB = cA + 1

            def doA(r):
                for k in range(GCHUNK // CHUNK):
                    pltpu.make_async_copy(
                        h_hbm.at[idxA.at[pl.ds(k * CHUNK, CHUNK)]],
                        rowsA.at[pl.ds(k * CHUNK, CHUNK)], semA).wait()

                @pl.when(cB < nchunks)
                def _prefB():
                    load_chunk(cB, idxB, rowsB, semB)
                return compute_chunk(cA, rowsA, r)

            def doB(r):
                for k in range(GCHUNK // CHUNK):
                    pltpu.make_async_copy(
                        h_hbm.at[idxB.at[pl.ds(k * CHUNK, CHUNK)]],
                        rowsB.at[pl.ds(k * CHUNK, CHUNK)], semB).wait()

                @pl.when(cB + 1 < nchunks)
                def _prefA():
                    load_chunk(cB + 1, idxA, rowsA, semA)
                return compute_chunk(cB, rowsB, r)

            r = lax.cond(cA < nchunks, doA, lambda r: r, r)
            r = lax.cond(cB < nchunks, doB, lambda r: r, r)
            return r
        lax.fori_loop(0, (nchunks + 1) // 2, pair_body, 0)

    @pl.when(cnt > CAP2)
    def _slow():
        def load_chunk(c, idxb, dstb, rows, sem):
            o = pl.multiple_of(base_in + c * GCHUNK, GCHUNK)
            pltpu.sync_copy(comp_src.at[pl.ds(o, GCHUNK)], idxb)
            for k in range(GCHUNK // CHUNK):
                pltpu.async_copy(h_hbm.at[idxb.at[pl.ds(k * CHUNK, CHUNK)]],
                                 rows.at[pl.ds(k * CHUNK, CHUNK)], sem)
            pltpu.sync_copy(comp_dst.at[pl.ds(o, GCHUNK)], dstb)

        def compute_chunk(rows, dstb):
            def group_body(g, _):
                row0 = g * 16
                dvec = dstb[pl.ds(row0, 16)]
                for e in range(16):
                    dsc = jnp.max(jnp.where(iota16 == e, dvec, 0))
                    for j in range(8):
                        a = acc[dsc, pl.ds(j * 16, 16)]
                        rr = rows[row0 + e, pl.ds(j * 16, 16)]
                        acc[dsc, pl.ds(j * 16, 16)] = jnp.maximum(a, rr)
                return _
            lax.fori_loop(0, GCHUNK // 16, group_body, None)

        load_chunk(0, idxA, dstA, rowsA, semA)

        def pair_body(c2, _):
            cA = c2 * 2
            cB = cA + 1

            @pl.when(cA < nchunks)
            def _doA():
                for k in range(GCHUNK // CHUNK):
                    pltpu.make_async_copy(
                        h_hbm.at[idxA.at[pl.ds(k * CHUNK, CHUNK)]],
                        rowsA.at[pl.ds(k * CHUNK, CHUNK)], semA).wait()

                @pl.when(cB < nchunks)
                def _prefB():
                    load_chunk(cB, idxB, dstB, rowsB, semB)
                compute_chunk(rowsA, dstA)

            @pl.when(cB < nchunks)
            def _doB():
                for k in range(GCHUNK // CHUNK):
                    pltpu.make_async_copy(
                        h_hbm.at[idxB.at[pl.ds(k * CHUNK, CHUNK)]],
                        rowsB.at[pl.ds(k * CHUNK, CHUNK)], semB).wait()

                @pl.when(cB + 1 < nchunks)
                def _prefA():
                    load_chunk(cB + 1, idxA, dstA, rowsA, semA)
                compute_chunk(rowsB, dstB)
            return _
        lax.fori_loop(0, (nchunks + 1) // 2, pair_body, None)

    def filt_body(r, _):
        for j in range(8):
            v = acc[r, pl.ds(j * 16, 16)]
            acc[r, pl.ds(j * 16, 16)] = jnp.where(v == -jnp.inf, 0.0, v)
        return _
    lax.fori_loop(0, RPW, filt_body, None)
    pltpu.sync_copy(acc.at[pl.ds(0, RPW)], out.at[pl.ds(w * RPW, RPW)])


def _sage_dense_body(agg_ref, h_ref, WlT_ref, bl_ref, WrT_ref, out_ref):
    acc = jnp.dot(agg_ref[...], WlT_ref[...], preferred_element_type=jnp.float32)
    acc += jnp.dot(h_ref[...], WrT_ref[...], preferred_element_type=jnp.float32)
    out_ref[...] = acc + bl_ref[...]


def _sage_dense(agg, h, WlT, bl, WrT):
    return pl.pallas_call(
        _sage_dense_body,
        grid=(N // ROW_BLK,),
        in_specs=[
            pl.BlockSpec((ROW_BLK, D), lambda i: (i, 0)),
            pl.BlockSpec((ROW_BLK, D), lambda i: (i, 0)),
            pl.BlockSpec((D, D), lambda i: (0, 0)),
            pl.BlockSpec((1, D), lambda i: (0, 0)),
            pl.BlockSpec((D, D), lambda i: (0, 0)),
        ],
        out_specs=pl.BlockSpec((ROW_BLK, D), lambda i: (i, 0)),
        out_shape=jax.ShapeDtypeStruct((N, D), jnp.float32),
    )(agg, h, WlT, bl, WrT)


def _mlp_body(h_ref, l1WT_ref, l1b_ref, l2WT_ref, l2b_ref, l3WT_ref, l3b_ref,
              out_ref):
    a = jnp.dot(h_ref[...], l1WT_ref[...], preferred_element_type=jnp.float32)
    a = jax.nn.relu(a + l1b_ref[...])
    a = jnp.dot(a, l2WT_ref[...], preferred_element_type=jnp.float32)
    a = jax.nn.relu(a + l2b_ref[...])
    a = jnp.dot(a, l3WT_ref[...], preferred_element_type=jnp.float32)
    out_ref[...] = jax.nn.sigmoid(a + l3b_ref[...])


def _mlp(h, l1WT, l1b, l2WT, l2b, l3WT, l3b):
    return pl.pallas_call(
        _mlp_body,
        grid=(N // ROW_BLK,),
        in_specs=[
            pl.BlockSpec((ROW_BLK, D), lambda i: (i, 0)),
            pl.BlockSpec((D, D), lambda i: (0, 0)),
            pl.BlockSpec((1, D), lambda i: (0, 0)),
            pl.BlockSpec((D, D), lambda i: (0, 0)),
            pl.BlockSpec((1, D), lambda i: (0, 0)),
            pl.BlockSpec((D, D), lambda i: (0, 0)),
            pl.BlockSpec((1, D), lambda i: (0, 0)),
        ],
        out_specs=pl.BlockSpec((ROW_BLK, D), lambda i: (i, 0)),
        out_shape=jax.ShapeDtypeStruct((N, D), jnp.float32),
    )(h, l1WT, l1b, l2WT, l2b, l3WT, l3b)


def kernel(x, edge_index, batch, W1l, b1l, W1r, W2l, b2l, W2r, W3l, b3l, W3r,
           l1W, l1b, l2W, l2b, l3W, l3b):
    src = edge_index[0]
    dst = edge_index[1]

    comp_src, comp_dst, counts, sorted_src, rowptr = _partition_edges(src, dst)


    h = x
    for Wl, bl, Wr in ((W1l, b1l, W1r), (W2l, b2l, W2r), (W3l, b3l, W3r)):
        agg = _segmax_sc(h, comp_src, comp_dst, counts, sorted_src, rowptr)[:N]
        h = _sage_dense(agg, h, Wl.T, bl[None, :], Wr.T)

    l2WT = jnp.pad(l2W.T, ((0, 0), (0, 64)))
    l2b_p = jnp.pad(l2b, (0, 64))[None, :]
    l3WT = jnp.pad(l3W.T, ((0, 64), (0, 122)))
    l3b_p = jnp.pad(l3b, (0, 122))[None, :]
    out = _mlp(h, l1W.T, l1b[None, :], l2WT, l2b_p, l3WT, l3b_p)
    return out[:, :6]


# speedup vs baseline: 1.3846x; 1.3846x over previous
import functools

import jax
import jax.numpy as jnp
from jax import lax
from jax.experimental import pallas as pl
from jax.experimental.pallas import tpu as pltpu
from jax.experimental.pallas import tpu_sc as plsc

N = 10000
E = 320000
D = 128
ROW_BLK = 2000

NC = 2
NS = 16
NW = NC * NS
RPW = 320
NPAD = NW * RPW
FLUSH = 8192
BUFCAP = FLUSH + 128
SCAN = 3200
NSCAN = E // SCAN
COMP_CAP = 40 * FLUSH
CHUNK = 128
NBIN = 336
CAP2 = 16384
CAP2B = CAP2 + FLUSH
RANK1 = 1

_mesh = plsc.VectorSubcoreMesh(
    core_axis_name="c", subcore_axis_name="s", num_cores=NC, num_subcores=NS)
_sc_params = pltpu.CompilerParams(needs_layout_passes=False)


def _wid():
    return lax.axis_index("s") * NC + lax.axis_index("c")


@functools.partial(
    pl.kernel,
    out_type=(
        jax.ShapeDtypeStruct((NW * COMP_CAP,), jnp.int32),
        jax.ShapeDtypeStruct((NW * COMP_CAP,), jnp.int32),
        jax.ShapeDtypeStruct((NW * 16,), jnp.int32),
        jax.ShapeDtypeStruct((NW * CAP2B,), jnp.int32),
        jax.ShapeDtypeStruct((NW * NBIN,), jnp.int32),
    ),
    mesh=_mesh,
    compiler_params=_sc_params,
    scratch_types=[
        pltpu.VMEM((SCAN,), jnp.int32),
        pltpu.VMEM((SCAN,), jnp.int32),
        pltpu.VMEM((SCAN,), jnp.int32),
        pltpu.VMEM((SCAN,), jnp.int32),
        pltpu.VMEM((BUFCAP,), jnp.int32),
        pltpu.VMEM((BUFCAP,), jnp.int32),
        pltpu.VMEM((NBIN,), jnp.int32),
        pltpu.VMEM((NBIN,), jnp.int32),
        pltpu.VMEM((CAP2B,), jnp.int32),
        pltpu.VMEM((16,), jnp.int32),
        pltpu.SemaphoreType.DMA,
        pltpu.SemaphoreType.DMA,
    ],
)
def _partition_edges(src_hbm, dst_hbm, comp_src, comp_dst, counts,
                     sorted_src, rowptr_out,
                     sbufA, sbufB, dbufA, dbufB, csrc, cdst,
                     hist, rowptr, sorted_buf, cbuf, semA, semB):
    w = _wid()
    lo = w * RPW
    base_out = w * COMP_CAP
    urpw = jnp.uint32(RPW)
    iota16 = lax.broadcasted_iota(jnp.int32, (16,), 0)

    zeros16 = jnp.zeros((16,), jnp.int32)
    trash16 = jnp.full((16,), RPW, jnp.int32)

    def init_body(k, _):
        for u in range(8):
            csrc[pl.ds((k * 8 + u) * 16, 16)] = zeros16
            cdst[pl.ds((k * 8 + u) * 16, 16)] = trash16
        return _
    lax.fori_loop(0, BUFCAP // 128, init_body, None)
    for k in range(NBIN // 16):
        hist[pl.ds(k * 16, 16)] = zeros16

    def start_load(c, sbuf, dbuf, sem):
        o = pl.multiple_of(c * SCAN, 8)
        pltpu.async_copy(src_hbm.at[pl.ds(o, SCAN)], sbuf, sem)
        pltpu.async_copy(dst_hbm.at[pl.ds(o, SCAN)], dbuf, sem)

    def wait_load(sbuf, dbuf, sem):
        pltpu.make_async_copy(src_hbm.at[pl.ds(0, SCAN)], sbuf, sem).wait()
        pltpu.make_async_copy(dst_hbm.at[pl.ds(0, SCAN)], dbuf, sem).wait()

    def scan_buf(sbuf, dbuf, pos, off):
        def vec_body(v, carry):
            pos, off = carry
            ds_, ss_, ms_, cs_ = [], [], [], []
            for u in range(8):
                b = v * 128 + u * 16
                d = dbuf[pl.ds(b, 16)]
                s = sbuf[pl.ds(b, 16)]
                dl = d - lo
                m = dl.astype(jnp.uint32) < urpw
                ds_.append(dl)
                ss_.append(s)
                ms_.append(m)
                cs_.append(jnp.sum(m.astype(jnp.int32)))
            for u in range(8):
                plsc.store_compressed(csrc.at[pl.ds(pos, 16)], ss_[u],
                                      mask=ms_[u])
                plsc.store_compressed(cdst.at[pl.ds(pos, 16)], ds_[u],
                                      mask=ms_[u])
                pos = pos + cs_[u]

            def do_flush(pos, off):
                o = pl.multiple_of(base_out + off, FLUSH)
                pltpu.sync_copy(csrc.at[pl.ds(0, FLUSH)],
                                comp_src.at[pl.ds(o, FLUSH)])
                pltpu.sync_copy(cdst.at[pl.ds(0, FLUSH)],
                                comp_dst.at[pl.ds(o, FLUSH)])
                for t in range(8):
                    csrc[pl.ds(t * 16, 16)] = csrc[pl.ds(FLUSH + t * 16, 16)]
                    cdst[pl.ds(t * 16, 16)] = cdst[pl.ds(FLUSH + t * 16, 16)]
                return pos - FLUSH, off + FLUSH

            return lax.cond(pos >= FLUSH, do_flush, lambda p, o: (p, o),
                            pos, off)
        return lax.fori_loop(0, SCAN // 128, vec_body, (pos, off))

    start_load(0, sbufA, dbufA, semA)

    def pair_body(c2, carry):
        pos, off = carry
        cB = c2 * 2 + 1
        wait_load(sbufA, dbufA, semA)
        start_load(cB, sbufB, dbufB, semB)
        pos, off = scan_buf(sbufA, dbufA, pos, off)
        wait_load(sbufB, dbufB, semB)

        @pl.when(cB + 1 < NSCAN)
        def _prefA():
            start_load(cB + 1, sbufA, dbufA, semA)
        pos, off = scan_buf(sbufB, dbufB, pos, off)
        return pos, off
    pos, off = lax.fori_loop(0, NSCAN // 2, pair_body, (0, 0))

    o = pl.multiple_of(base_out + off, FLUSH)
    pltpu.sync_copy(csrc.at[pl.ds(0, FLUSH)],
                    comp_src.at[pl.ds(o, FLUSH)])
    pltpu.sync_copy(cdst.at[pl.ds(0, FLUSH)],
                    comp_dst.at[pl.ds(o, FLUSH)])
    cnt = off + pos
    cbuf[pl.ds(0, 16)] = jnp.full((16,), cnt, jnp.int32)
    pltpu.sync_copy(cbuf, counts.at[pl.ds(pl.multiple_of(w * 16, 16), 16)])

    @pl.when(cnt <= CAP2)
    def _sort():
        nh = (cnt + (SCAN - 1)) // SCAN

        def hist_chunk(hc, _):
            o = pl.multiple_of(base_out + hc * SCAN, 8)
            pltpu.sync_copy(comp_dst.at[pl.ds(o, SCAN)], dbufA)
            e0 = hc * SCAN

            def hvec(v, _):
                d = dbufA[pl.ds(v * 16, 16)]
                valid = (iota16 + (e0 + v * 16)) < cnt
                d = jnp.where(valid, d, RPW)
                rank, lastm = plsc.scan_count(d, mask=valid)
                plsc.addupdate_scatter(hist, [d], rank + (1 - RANK1),
                                       mask=lastm)
                return _
            lax.fori_loop(0, SCAN // 16, hvec, None)
            return _
        lax.fori_loop(0, nh, hist_chunk, None)

        def pfx(k, carry):
            v = hist[pl.ds(k * 16, 16)]
            incl = plsc.cumsum(v)
            rowptr[pl.ds(k * 16, 16)] = incl - v + carry
            return carry + jnp.max(incl)
        lax.fori_loop(0, NBIN // 16, pfx, 0)
        pltpu.sync_copy(
            rowptr, rowptr_out.at[pl.ds(pl.multiple_of(w * NBIN, 8), NBIN)])
        for k in range(NBIN // 16):
            hist[pl.ds(k * 16, 16)] = rowptr[pl.ds(k * 16, 16)]

        def zinit(k, _):
            sorted_buf[pl.ds(k * 16, 16)] = zeros16
            return _
        lax.fori_loop(0, (cnt + CHUNK + 15) // 16, zinit, None)

        def scat_chunk(hc, _):
            o = pl.multiple_of(base_out + hc * SCAN, 8)
            pltpu.sync_copy(comp_dst.at[pl.ds(o, SCAN)], dbufA)
            pltpu.sync_copy(comp_src.at[pl.ds(o, SCAN)], sbufA)
            e0 = hc * SCAN

            def svec(v, _):
                d = dbufA[pl.ds(v * 16, 16)]
                s = sbufA[pl.ds(v * 16, 16)]
                valid = (iota16 + (e0 + v * 16)) < cnt
                d = jnp.where(valid, d, RPW)
                rank, lastm = plsc.scan_count(d, mask=valid)
                base = plsc.load_gather(hist, [d])
                plsc.store_scatter(sorted_buf, [base + (rank - RANK1)], s,
                                   mask=valid)
                plsc.addupdate_scatter(hist, [d], rank + (1 - RANK1),
                                       mask=lastm)
                return _
            lax.fori_loop(0, SCAN // 16, svec, None)
            return _
        lax.fori_loop(0, nh, scat_chunk, None)

        def pad_body(k, _):
            sorted_buf[pl.ds(cnt + k * 16, 16)] = zeros16
            return _
        lax.fori_loop(0, CHUNK // 16, pad_body, None)

        npiece = (cnt + CHUNK + (FLUSH - 1)) // FLUSH

        def piece(p, _):
            src_o = pl.multiple_of(p * FLUSH, 8)
            dst_o = pl.multiple_of(w * CAP2B + p * FLUSH, 8)
            pltpu.sync_copy(sorted_buf.at[pl.ds(src_o, FLUSH)],
                            sorted_src.at[pl.ds(dst_o, FLUSH)])
            return _
        lax.fori_loop(0, npiece, piece, None)


@functools.partial(
    pl.kernel,
    out_type=jax.ShapeDtypeStruct((NPAD, D), jnp.float32),
    mesh=_mesh,
    compiler_params=_sc_params,
    scratch_types=[
        pltpu.VMEM((CAP2 + FLUSH,), jnp.int32),
        pltpu.VMEM((CHUNK,), jnp.int32),
        pltpu.VMEM((CHUNK,), jnp.int32),
        pltpu.VMEM((CHUNK,), jnp.int32),
        pltpu.VMEM((CHUNK,), jnp.int32),
        pltpu.VMEM((CHUNK, D), jnp.float32),
        pltpu.VMEM((CHUNK, D), jnp.float32),
        pltpu.VMEM((RPW + 4, D), jnp.float32),
        pltpu.VMEM((NBIN,), jnp.int32),
        pltpu.SMEM((NBIN,), jnp.int32),
        pltpu.VMEM((16,), jnp.int32),
        pltpu.SemaphoreType.DMA,
        pltpu.SemaphoreType.DMA,
    ],
)
def _segmax_sc(h_hbm, comp_src, comp_dst, counts, sorted_src, rowptr_hbm, out,
               idxr, idxA, idxB, dstA, dstB, rowsA, rowsB, acc, rpv, rps,
               cbuf, semA, semB):
    w = _wid()
    base_in = w * COMP_CAP
    base_s = w * CAP2B

    pltpu.sync_copy(counts.at[pl.ds(pl.multiple_of(w * 16, 16), 16)], cbuf)
    cnt = jnp.max(cbuf[pl.ds(0, 16)])
    nchunks = (cnt + (CHUNK - 1)) // CHUNK

    neg16 = jnp.full((16,), -jnp.inf, jnp.float32)
    iota16 = lax.broadcasted_iota(jnp.int32, (16,), 0)

    def init_body(k, _):
        for u in range(4):
            for j in range(8):
                acc[k * 4 + u, pl.ds(j * 16, 16)] = neg16
        return _
    lax.fori_loop(0, (RPW + 4) // 4, init_body, None)

    @pl.when(cnt <= CAP2)
    def _fast():
        pltpu.sync_copy(
            rowptr_hbm.at[pl.ds(pl.multiple_of(w * NBIN, 8), NBIN)], rpv)

        npiece = (cnt + CHUNK + (FLUSH - 1)) // FLUSH

        def ipiece(p, _):
            o = pl.multiple_of(p * FLUSH, 8)
            pltpu.sync_copy(sorted_src.at[pl.ds(pl.multiple_of(
                base_s + p * FLUSH, 8), FLUSH)], idxr.at[pl.ds(o, FLUSH)])
            return _
        lax.fori_loop(0, npiece, ipiece, None)

        def rp_body(k, _):
            vec = rpv[pl.ds(k * 16, 16)]
            for e in range(16):
                rps[k * 16 + e] = jnp.max(jnp.where(iota16 == e, vec, 0))
            return _
        lax.fori_loop(0, NBIN // 16, rp_body, None)

        def load_chunk(c, rows, sem):
            o = pl.multiple_of(c * CHUNK, 8)
            pltpu.async_copy(h_hbm.at[idxr.at[pl.ds(o, CHUNK)]], rows, sem)

        def compute_chunk(c, rows, r):
            e0 = c * CHUNK
            e1 = e0 + CHUNK

            def run_cond(carry):
                r, done = carry
                return jnp.logical_and(
                    jnp.logical_not(done),
                    jnp.logical_and(r < RPW, rps[r] < e1))

            def run_body(carry):
                r, done = carry
                s0 = rps[r]
                s1 = rps[r + 1]
                lo_e = jnp.maximum(s0, e0) - e0
                hi_e = jnp.minimum(s1, e1) - e0

                @pl.when(hi_e > lo_e)
                def _accum():
                    vals = [acc[r, pl.ds(j * 16, 16)] for j in range(8)]

                    def emax2(i, vals):
                        e = lo_e + i * 2
                        return [jnp.maximum(
                                    jnp.maximum(vals[j],
                                                rows[e, pl.ds(j * 16, 16)]),
                                    rows[e + 1, pl.ds(j * 16, 16)])
                                for j in range(8)]
                    n2 = (hi_e - lo_e) // 2
                    vals = lax.fori_loop(0, n2, emax2, vals)

                    def vtail(vals):
                        e = lo_e + n2 * 2
                        return [jnp.maximum(vals[j],
                                            rows[e, pl.ds(j * 16, 16)])
                                for j in range(8)]
                    vals = lax.cond((hi_e - lo_e) % 2 == 1, vtail,
                                    lambda v: v, vals)
                    for j in range(8):
                        acc[r, pl.ds(j * 16, 16)] = vals[j]

                adv = s1 <= e1
                return jnp.where(adv, r + 1, r), jnp.logical_not(adv)

            r, _done = lax.while_loop(run_cond, run_body, (r, False))
            return r

        @pl.when(nchunks > 0)
        def _prologue():
            load_chunk(0, rowsA, semA)

        def pair_body(c2, r):
            cA = c2 * 2
            cB = cA + 1

            def doA(r):
                pltpu.make_async_copy(
                    h_hbm.at[idxr.at[pl.ds(0, CHUNK)]], rowsA, semA).wait()

                @pl.when(cB < nchunks)
                def _prefB():
                    load_chunk(cB, rowsB, semB)
                return compute_chunk(cA, rowsA, r)

            def doB(r):
                pltpu.make_async_copy(
                    h_hbm.at[idxr.at[pl.ds(0, CHUNK)]], rowsB, semB).wait()

                @pl.when(cB + 1 < nchunks)
                def _prefA():
                    load_chunk(cB + 1, rowsA, semA)
                return compute_chunk(cB, rowsB, r)

            r = lax.cond(cA < nchunks, doA, lambda r: r, r)
            r = lax.cond(cB < nchunks, doB, lambda r: r, r)
            return r
        lax.fori_loop(0, (nchunks + 1) // 2, pair_body, 0)

    @pl.when(cnt > CAP2)
    def _slow():
        def load_chunk(c, idxb, dstb, rows, sem):
            o = pl.multiple_of(base_in + c * CHUNK, CHUNK)
            pltpu.sync_copy(comp_src.at[pl.ds(o, CHUNK)], idxb)
            pltpu.async_copy(h_hbm.at[idxb], rows, sem)
            pltpu.sync_copy(comp_dst.at[pl.ds(o, CHUNK)], dstb)

        def compute_chunk(rows, dstb):
            def group_body(g, _):
                row0 = g * 16
                dvec = dstb[pl.ds(row0, 16)]
                for e in range(16):
                    dsc = jnp.max(jnp.where(iota16 == e, dvec, 0))
                    for j in range(8):
                        a = acc[dsc, pl.ds(j * 16, 16)]
                        rr = rows[row0 + e, pl.ds(j * 16, 16)]
                        acc[dsc, pl.ds(j * 16, 16)] = jnp.maximum(a, rr)
                return _
            lax.fori_loop(0, CHUNK // 16, group_body, None)

        load_chunk(0, idxA, dstA, rowsA, semA)

        def pair_body(c2, _):
            cA = c2 * 2
            cB = cA + 1

            @pl.when(cA < nchunks)
            def _doA():
                pltpu.make_async_copy(h_hbm.at[idxA], rowsA, semA).wait()

                @pl.when(cB < nchunks)
                def _prefB():
                    load_chunk(cB, idxB, dstB, rowsB, semB)
                compute_chunk(rowsA, dstA)

            @pl.when(cB < nchunks)
            def _doB():
                pltpu.make_async_copy(h_hbm.at[idxB], rowsB, semB).wait()

                @pl.when(cB + 1 < nchunks)
                def _prefA():
                    load_chunk(cB + 1, idxA, dstA, rowsA, semA)
                compute_chunk(rowsB, dstB)
            return _
        lax.fori_loop(0, (nchunks + 1) // 2, pair_body, None)

    def filt_body(r, _):
        for j in range(8):
            v = acc[r, pl.ds(j * 16, 16)]
            acc[r, pl.ds(j * 16, 16)] = jnp.where(v == -jnp.inf, 0.0, v)
        return _
    lax.fori_loop(0, RPW, filt_body, None)
    pltpu.sync_copy(acc.at[pl.ds(0, RPW)], out.at[pl.ds(w * RPW, RPW)])


def _sage_dense_body(agg_ref, h_ref, WlT_ref, bl_ref, WrT_ref, out_ref):
    acc = jnp.dot(agg_ref[...], WlT_ref[...], preferred_element_type=jnp.float32)
    acc += jnp.dot(h_ref[...], WrT_ref[...], preferred_element_type=jnp.float32)
    out_ref[...] = acc + bl_ref[...]


def _sage_dense(agg, h, WlT, bl, WrT):
    return pl.pallas_call(
        _sage_dense_body,
        grid=(N // ROW_BLK,),
        in_specs=[
            pl.BlockSpec((ROW_BLK, D), lambda i: (i, 0)),
            pl.BlockSpec((ROW_BLK, D), lambda i: (i, 0)),
            pl.BlockSpec((D, D), lambda i: (0, 0)),
            pl.BlockSpec((1, D), lambda i: (0, 0)),
            pl.BlockSpec((D, D), lambda i: (0, 0)),
        ],
        out_specs=pl.BlockSpec((ROW_BLK, D), lambda i: (i, 0)),
        out_shape=jax.ShapeDtypeStruct((N, D), jnp.float32),
    )(agg, h, WlT, bl, WrT)


def _mlp_body(h_ref, l1WT_ref, l1b_ref, l2WT_ref, l2b_ref, l3WT_ref, l3b_ref,
              out_ref):
    a = jnp.dot(h_ref[...], l1WT_ref[...], preferred_element_type=jnp.float32)
    a = jax.nn.relu(a + l1b_ref[...])
    a = jnp.dot(a, l2WT_ref[...], preferred_element_type=jnp.float32)
    a = jax.nn.relu(a + l2b_ref[...])
    a = jnp.dot(a, l3WT_ref[...], preferred_element_type=jnp.float32)
    out_ref[...] = jax.nn.sigmoid(a + l3b_ref[...])


def _mlp(h, l1WT, l1b, l2WT, l2b, l3WT, l3b):
    return pl.pallas_call(
        _mlp_body,
        grid=(N // ROW_BLK,),
        in_specs=[
            pl.BlockSpec((ROW_BLK, D), lambda i: (i, 0)),
            pl.BlockSpec((D, D), lambda i: (0, 0)),
            pl.BlockSpec((1, D), lambda i: (0, 0)),
            pl.BlockSpec((D, D), lambda i: (0, 0)),
            pl.BlockSpec((1, D), lambda i: (0, 0)),
            pl.BlockSpec((D, D), lambda i: (0, 0)),
            pl.BlockSpec((1, D), lambda i: (0, 0)),
        ],
        out_specs=pl.BlockSpec((ROW_BLK, D), lambda i: (i, 0)),
        out_shape=jax.ShapeDtypeStruct((N, D), jnp.float32),
    )(h, l1WT, l1b, l2WT, l2b, l3WT, l3b)


def kernel(x, edge_index, batch, W1l, b1l, W1r, W2l, b2l, W2r, W3l, b3l, W3r,
           l1W, l1b, l2W, l2b, l3W, l3b):
    src = edge_index[0]
    dst = edge_index[1]

    comp_src, comp_dst, counts, sorted_src, rowptr = _partition_edges(src, dst)


    h = x
    for Wl, bl, Wr in ((W1l, b1l, W1r), (W2l, b2l, W2r), (W3l, b3l, W3r)):
        agg = _segmax_sc(h, comp_src, comp_dst, counts, sorted_src, rowptr)[:N]
        h = _sage_dense(agg, h, Wl.T, bl[None, :], Wr.T)

    l2WT = jnp.pad(l2W.T, ((0, 0), (0, 64)))
    l2b_p = jnp.pad(l2b, (0, 64))[None, :]
    l3WT = jnp.pad(l3W.T, ((0, 64), (0, 122)))
    l3b_p = jnp.pad(l3b, (0, 122))[None, :]
    out = _mlp(h, l1W.T, l1b[None, :], l2WT, l2b_p, l3WT, l3b_p)
    return out[:, :6]


# speedup vs baseline: 1.3977x; 1.0095x over previous
import functools

import jax
import jax.numpy as jnp
from jax import lax
from jax.experimental import pallas as pl
from jax.experimental.pallas import tpu as pltpu
from jax.experimental.pallas import tpu_sc as plsc

N = 10000
E = 320000
D = 128
ROW_BLK = 2000

NC = 2
NS = 16
NW = NC * NS
RPW = 320
NPAD = NW * RPW
FLUSH = 8192
BUFCAP = FLUSH + 128
SCAN = 3200
NSCAN = E // SCAN
COMP_CAP = 40 * FLUSH
CHUNK = 128
NBIN = 336
CAP2 = 16384
CAP2B = CAP2 + FLUSH
RANK1 = 1

_mesh = plsc.VectorSubcoreMesh(
    core_axis_name="c", subcore_axis_name="s", num_cores=NC, num_subcores=NS)
_sc_params = pltpu.CompilerParams(needs_layout_passes=False)


def _wid():
    return lax.axis_index("s") * NC + lax.axis_index("c")


@functools.partial(
    pl.kernel,
    out_type=(
        jax.ShapeDtypeStruct((NW * COMP_CAP,), jnp.int32),
        jax.ShapeDtypeStruct((NW * COMP_CAP,), jnp.int32),
        jax.ShapeDtypeStruct((NW * 16,), jnp.int32),
        jax.ShapeDtypeStruct((NW * CAP2B,), jnp.int32),
        jax.ShapeDtypeStruct((NW * NBIN,), jnp.int32),
    ),
    mesh=_mesh,
    compiler_params=_sc_params,
    scratch_types=[
        pltpu.VMEM((SCAN,), jnp.int32),
        pltpu.VMEM((SCAN,), jnp.int32),
        pltpu.VMEM((SCAN,), jnp.int32),
        pltpu.VMEM((SCAN,), jnp.int32),
        pltpu.VMEM((BUFCAP,), jnp.int32),
        pltpu.VMEM((BUFCAP,), jnp.int32),
        pltpu.VMEM((NBIN,), jnp.int32),
        pltpu.VMEM((NBIN,), jnp.int32),
        pltpu.VMEM((CAP2B,), jnp.int32),
        pltpu.VMEM((16,), jnp.int32),
        pltpu.SemaphoreType.DMA,
        pltpu.SemaphoreType.DMA,
    ],
)
def _partition_edges(src_hbm, dst_hbm, comp_src, comp_dst, counts,
                     sorted_src, rowptr_out,
                     sbufA, sbufB, dbufA, dbufB, csrc, cdst,
                     hist, rowptr, sorted_buf, cbuf, semA, semB):
    w = _wid()
    lo = w * RPW
    base_out = w * COMP_CAP
    urpw = jnp.uint32(RPW)
    iota16 = lax.broadcasted_iota(jnp.int32, (16,), 0)

    zeros16 = jnp.zeros((16,), jnp.int32)
    trash16 = jnp.full((16,), RPW, jnp.int32)

    def init_body(k, _):
        for u in range(8):
            csrc[pl.ds((k * 8 + u) * 16, 16)] = zeros16
            cdst[pl.ds((k * 8 + u) * 16, 16)] = trash16
        return _
    lax.fori_loop(0, BUFCAP // 128, init_body, None)
    for k in range(NBIN // 16):
        hist[pl.ds(k * 16, 16)] = zeros16

    def start_load(c, sbuf, dbuf, sem):
        o = pl.multiple_of(c * SCAN, 8)
        pltpu.async_copy(src_hbm.at[pl.ds(o, SCAN)], sbuf, sem)
        pltpu.async_copy(dst_hbm.at[pl.ds(o, SCAN)], dbuf, sem)

    def wait_load(sbuf, dbuf, sem):
        pltpu.make_async_copy(src_hbm.at[pl.ds(0, SCAN)], sbuf, sem).wait()
        pltpu.make_async_copy(dst_hbm.at[pl.ds(0, SCAN)], dbuf, sem).wait()

    def scan_buf(sbuf, dbuf, pos, off):
        def vec_body(v, carry):
            pos, off = carry
            ds_, ss_, ms_, cs_ = [], [], [], []
            for u in range(8):
                b = v * 128 + u * 16
                d = dbuf[pl.ds(b, 16)]
                s = sbuf[pl.ds(b, 16)]
                dl = d - lo
                m = dl.astype(jnp.uint32) < urpw
                ds_.append(dl)
                ss_.append(s)
                ms_.append(m)
                cs_.append(jnp.sum(m.astype(jnp.int32)))
            for u in range(8):
                plsc.store_compressed(csrc.at[pl.ds(pos, 16)], ss_[u],
                                      mask=ms_[u])
                plsc.store_compressed(cdst.at[pl.ds(pos, 16)], ds_[u],
                                      mask=ms_[u])
                pos = pos + cs_[u]

            def do_flush(pos, off):
                o = pl.multiple_of(base_out + off, FLUSH)
                pltpu.sync_copy(csrc.at[pl.ds(0, FLUSH)],
                                comp_src.at[pl.ds(o, FLUSH)])
                pltpu.sync_copy(cdst.at[pl.ds(0, FLUSH)],
                                comp_dst.at[pl.ds(o, FLUSH)])
                for t in range(8):
                    csrc[pl.ds(t * 16, 16)] = csrc[pl.ds(FLUSH + t * 16, 16)]
                    cdst[pl.ds(t * 16, 16)] = cdst[pl.ds(FLUSH + t * 16, 16)]
                return pos - FLUSH, off + FLUSH

            return lax.cond(pos >= FLUSH, do_flush, lambda p, o: (p, o),
                            pos, off)
        return lax.fori_loop(0, SCAN // 128, vec_body, (pos, off))

    start_load(0, sbufA, dbufA, semA)

    def pair_body(c2, carry):
        pos, off = carry
        cB = c2 * 2 + 1
        wait_load(sbufA, dbufA, semA)
        start_load(cB, sbufB, dbufB, semB)
        pos, off = scan_buf(sbufA, dbufA, pos, off)
        wait_load(sbufB, dbufB, semB)

        @pl.when(cB + 1 < NSCAN)
        def _prefA():
            start_load(cB + 1, sbufA, dbufA, semA)
        pos, off = scan_buf(sbufB, dbufB, pos, off)
        return pos, off
    pos, off = lax.fori_loop(0, NSCAN // 2, pair_body, (0, 0))

    o = pl.multiple_of(base_out + off, FLUSH)
    pltpu.sync_copy(csrc.at[pl.ds(0, FLUSH)],
                    comp_src.at[pl.ds(o, FLUSH)])
    pltpu.sync_copy(cdst.at[pl.ds(0, FLUSH)],
                    comp_dst.at[pl.ds(o, FLUSH)])
    cnt = off + pos
    cbuf[pl.ds(0, 16)] = jnp.full((16,), cnt, jnp.int32)
    pltpu.sync_copy(cbuf, counts.at[pl.ds(pl.multiple_of(w * 16, 16), 16)])

    @pl.when(cnt <= CAP2)
    def _sort():
        nh = (cnt + (SCAN - 1)) // SCAN

        def hist_chunk(hc, _):
            o = pl.multiple_of(base_out + hc * SCAN, 8)
            pltpu.sync_copy(comp_dst.at[pl.ds(o, SCAN)], dbufA)
            e0 = hc * SCAN

            def hvec(v, _):
                d = dbufA[pl.ds(v * 16, 16)]
                valid = (iota16 + (e0 + v * 16)) < cnt
                d = jnp.where(valid, d, RPW)
                rank, lastm = plsc.scan_count(d, mask=valid)
                plsc.addupdate_scatter(hist, [d], rank + (1 - RANK1),
                                       mask=lastm)
                return _
            lax.fori_loop(0, SCAN // 16, hvec, None)
            return _
        lax.fori_loop(0, nh, hist_chunk, None)

        def pfx(k, carry):
            v = hist[pl.ds(k * 16, 16)]
            incl = plsc.cumsum(v)
            rowptr[pl.ds(k * 16, 16)] = incl - v + carry
            return carry + jnp.max(incl)
        lax.fori_loop(0, NBIN // 16, pfx, 0)
        pltpu.sync_copy(
            rowptr, rowptr_out.at[pl.ds(pl.multiple_of(w * NBIN, 8), NBIN)])
        for k in range(NBIN // 16):
            hist[pl.ds(k * 16, 16)] = rowptr[pl.ds(k * 16, 16)]

        def zinit(k, _):
            sorted_buf[pl.ds(k * 16, 16)] = zeros16
            return _
        lax.fori_loop(0, (cnt + CHUNK + 15) // 16, zinit, None)

        def scat_chunk(hc, _):
            o = pl.multiple_of(base_out + hc * SCAN, 8)
            pltpu.sync_copy(comp_dst.at[pl.ds(o, SCAN)], dbufA)
            pltpu.sync_copy(comp_src.at[pl.ds(o, SCAN)], sbufA)
            e0 = hc * SCAN

            def svec(v, _):
                d = dbufA[pl.ds(v * 16, 16)]
                s = sbufA[pl.ds(v * 16, 16)]
                valid = (iota16 + (e0 + v * 16)) < cnt
                d = jnp.where(valid, d, RPW)
                rank, lastm = plsc.scan_count(d, mask=valid)
                base = plsc.load_gather(hist, [d])
                plsc.store_scatter(sorted_buf, [base + (rank - RANK1)], s,
                                   mask=valid)
                plsc.addupdate_scatter(hist, [d], rank + (1 - RANK1),
                                       mask=lastm)
                return _
            lax.fori_loop(0, SCAN // 16, svec, None)
            return _
        lax.fori_loop(0, nh, scat_chunk, None)

        def pad_body(k, _):
            sorted_buf[pl.ds(cnt + k * 16, 16)] = zeros16
            return _
        lax.fori_loop(0, CHUNK // 16, pad_body, None)

        npiece = (cnt + CHUNK + (FLUSH - 1)) // FLUSH

        def piece(p, _):
            src_o = pl.multiple_of(p * FLUSH, 8)
            dst_o = pl.multiple_of(w * CAP2B + p * FLUSH, 8)
            pltpu.sync_copy(sorted_buf.at[pl.ds(src_o, FLUSH)],
                            sorted_src.at[pl.ds(dst_o, FLUSH)])
            return _
        lax.fori_loop(0, npiece, piece, None)


@functools.partial(
    pl.kernel,
    out_type=jax.ShapeDtypeStruct((NPAD, D), jnp.float32),
    mesh=_mesh,
    compiler_params=_sc_params,
    scratch_types=[
        pltpu.VMEM((CAP2 + FLUSH,), jnp.int32),
        pltpu.VMEM((CHUNK,), jnp.int32),
        pltpu.VMEM((CHUNK,), jnp.int32),
        pltpu.VMEM((CHUNK,), jnp.int32),
        pltpu.VMEM((CHUNK,), jnp.int32),
        pltpu.VMEM((CHUNK, D), jnp.float32),
        pltpu.VMEM((CHUNK, D), jnp.float32),
        pltpu.VMEM((RPW + 4, D), jnp.float32),
        pltpu.VMEM((NBIN,), jnp.int32),
        pltpu.SMEM((NBIN,), jnp.int32),
        pltpu.VMEM((16,), jnp.int32),
        pltpu.SemaphoreType.DMA,
        pltpu.SemaphoreType.DMA,
    ],
)
def _segmax_sc(h_hbm, comp_src, comp_dst, counts, sorted_src, rowptr_hbm, out,
               idxr, idxA, idxB, dstA, dstB, rowsA, rowsB, acc, rpv, rps,
               cbuf, semA, semB):
    w = _wid()
    base_in = w * COMP_CAP
    base_s = w * CAP2B

    pltpu.sync_copy(counts.at[pl.ds(pl.multiple_of(w * 16, 16), 16)], cbuf)
    cnt = jnp.max(cbuf[pl.ds(0, 16)])
    nchunks = (cnt + (CHUNK - 1)) // CHUNK

    neg16 = jnp.full((16,), -jnp.inf, jnp.float32)
    iota16 = lax.broadcasted_iota(jnp.int32, (16,), 0)

    def init_body(k, _):
        for u in range(4):
            for j in range(8):
                acc[k * 4 + u, pl.ds(j * 16, 16)] = neg16
        return _
    lax.fori_loop(0, (RPW + 4) // 4, init_body, None)

    @pl.when(cnt <= CAP2)
    def _fast():
        pltpu.sync_copy(
            rowptr_hbm.at[pl.ds(pl.multiple_of(w * NBIN, 8), NBIN)], rpv)

        npiece = (cnt + CHUNK + (FLUSH - 1)) // FLUSH

        def ipiece(p, _):
            o = pl.multiple_of(p * FLUSH, 8)
            pltpu.sync_copy(sorted_src.at[pl.ds(pl.multiple_of(
                base_s + p * FLUSH, 8), FLUSH)], idxr.at[pl.ds(o, FLUSH)])
            return _
        lax.fori_loop(0, npiece, ipiece, None)

        def rp_body(k, _):
            vec = rpv[pl.ds(k * 16, 16)]
            for e in range(16):
                rps[k * 16 + e] = jnp.max(jnp.where(iota16 == e, vec, 0))
            return _
        lax.fori_loop(0, NBIN // 16, rp_body, None)

        def load_chunk(c, rows, sem):
            o = pl.multiple_of(c * CHUNK, 8)
            pltpu.async_copy(h_hbm.at[idxr.at[pl.ds(o, CHUNK)]], rows, sem)

        def compute_chunk(c, rows, r):
            e0 = c * CHUNK
            e1 = e0 + CHUNK

            def run_cond(carry):
                r, done = carry
                return jnp.logical_and(
                    jnp.logical_not(done),
                    jnp.logical_and(r < RPW, rps[r] < e1))

            def run_body(carry):
                r, done = carry
                s0 = rps[r]
                s1 = rps[r + 1]
                lo_e = jnp.maximum(s0, e0) - e0
                hi_e = jnp.minimum(s1, e1) - e0

                @pl.when(hi_e > lo_e)
                def _accum():
                    vals = [acc[r, pl.ds(j * 16, 16)] for j in range(8)]

                    def emax2(i, vals):
                        e = lo_e + i * 2
                        return [jnp.maximum(
                                    jnp.maximum(vals[j],
                                                rows[e, pl.ds(j * 16, 16)]),
                                    rows[e + 1, pl.ds(j * 16, 16)])
                                for j in range(8)]
                    n2 = (hi_e - lo_e) // 2
                    vals = lax.fori_loop(0, n2, emax2, vals)

                    def vtail(vals):
                        e = lo_e + n2 * 2
                        return [jnp.maximum(vals[j],
                                            rows[e, pl.ds(j * 16, 16)])
                                for j in range(8)]
                    vals = lax.cond((hi_e - lo_e) % 2 == 1, vtail,
                                    lambda v: v, vals)
                    for j in range(8):
                        acc[r, pl.ds(j * 16, 16)] = vals[j]

                adv = s1 <= e1
                return jnp.where(adv, r + 1, r), jnp.logical_not(adv)

            r, _done = lax.while_loop(run_cond, run_body, (r, False))
            return r

        @pl.when(nchunks > 0)
        def _prologue():
            load_chunk(0, rowsA, semA)

        def pair_body(c2, r):
            cA = c2 * 2
            cB = cA + 1

            def doA(r):
                pltpu.make_async_copy(
                    h_hbm.at[idxr.at[pl.ds(0, CHUNK)]], rowsA, semA).wait()

                @pl.when(cB < nchunks)
                def _prefB():
                    load_chunk(cB, rowsB, semB)
                return compute_chunk(cA, rowsA, r)

            def doB(r):
                pltpu.make_async_copy(
                    h_hbm.at[idxr.at[pl.ds(0, CHUNK)]], rowsB, semB).wait()

                @pl.when(cB + 1 < nchunks)
                def _prefA():
                    load_chunk(cB + 1, rowsA, semA)
                return compute_chunk(cB, rowsB, r)

            r = lax.cond(cA < nchunks, doA, lambda r: r, r)
            r = lax.cond(cB < nchunks, doB, lambda r: r, r)
            return r
        lax.fori_loop(0, (nchunks + 1) // 2, pair_body, 0)

    @pl.when(cnt > CAP2)
    def _slow():
        def load_chunk(c, idxb, dstb, rows, sem):
            o = pl.multiple_of(base_in + c * CHUNK, CHUNK)
            pltpu.sync_copy(comp_src.at[pl.ds(o, CHUNK)], idxb)
            pltpu.async_copy(h_hbm.at[idxb], rows, sem)
            pltpu.sync_copy(comp_dst.at[pl.ds(o, CHUNK)], dstb)

        def compute_chunk(rows, dstb):
            def group_body(g, _):
                row0 = g * 16
                dvec = dstb[pl.ds(row0, 16)]
                for e in range(16):
                    dsc = jnp.max(jnp.where(iota16 == e, dvec, 0))
                    for j in range(8):
                        a = acc[dsc, pl.ds(j * 16, 16)]
                        rr = rows[row0 + e, pl.ds(j * 16, 16)]
                        acc[dsc, pl.ds(j * 16, 16)] = jnp.maximum(a, rr)
                return _
            lax.fori_loop(0, CHUNK // 16, group_body, None)

        load_chunk(0, idxA, dstA, rowsA, semA)

        def pair_body(c2, _):
            cA = c2 * 2
            cB = cA + 1

            @pl.when(cA < nchunks)
            def _doA():
                pltpu.make_async_copy(h_hbm.at[idxA], rowsA, semA).wait()

                @pl.when(cB < nchunks)
                def _prefB():
                    load_chunk(cB, idxB, dstB, rowsB, semB)
                compute_chunk(rowsA, dstA)

            @pl.when(cB < nchunks)
            def _doB():
                pltpu.make_async_copy(h_hbm.at[idxB], rowsB, semB).wait()

                @pl.when(cB + 1 < nchunks)
                def _prefA():
                    load_chunk(cB + 1, idxA, dstA, rowsA, semA)
                compute_chunk(rowsB, dstB)
            return _
        lax.fori_loop(0, (nchunks + 1) // 2, pair_body, None)

    def filt_body(r, _):
        for j in range(8):
            v = acc[r, pl.ds(j * 16, 16)]
            acc[r, pl.ds(j * 16, 16)] = jnp.where(v == -jnp.inf, 0.0, v)
        return _
    lax.fori_loop(0, RPW, filt_body, None)
    pltpu.sync_copy(acc.at[pl.ds(0, RPW)], out.at[pl.ds(w * RPW, RPW)])


def _sage_dense_body(agg_ref, h_ref, WlT_ref, bl_ref, WrT_ref, out_ref):
    acc = jnp.dot(agg_ref[...], WlT_ref[...], preferred_element_type=jnp.float32)
    acc += jnp.dot(h_ref[...], WrT_ref[...], preferred_element_type=jnp.float32)
    out_ref[...] = acc + bl_ref[...]


def _sage_dense(agg, h, WlT, bl, WrT):
    return pl.pallas_call(
        _sage_dense_body,
        grid=(N // ROW_BLK,),
        in_specs=[
            pl.BlockSpec((ROW_BLK, D), lambda i: (i, 0)),
            pl.BlockSpec((ROW_BLK, D), lambda i: (i, 0)),
            pl.BlockSpec((D, D), lambda i: (0, 0)),
            pl.BlockSpec((1, D), lambda i: (0, 0)),
            pl.BlockSpec((D, D), lambda i: (0, 0)),
        ],
        out_specs=pl.BlockSpec((ROW_BLK, D), lambda i: (i, 0)),
        out_shape=jax.ShapeDtypeStruct((N, D), jnp.float32),
    )(agg, h, WlT, bl, WrT)


def _mlp_body(agg_ref, h_ref, WlT_ref, bl_ref, WrT_ref,
              l1WT_ref, l1b_ref, l2WT_ref, l2b_ref, l3WT_ref, l3b_ref,
              out_ref):
    hh = jnp.dot(agg_ref[...], WlT_ref[...], preferred_element_type=jnp.float32)
    hh += jnp.dot(h_ref[...], WrT_ref[...], preferred_element_type=jnp.float32)
    hh += bl_ref[...]
    a = jnp.dot(hh, l1WT_ref[...], preferred_element_type=jnp.float32)
    a = jax.nn.relu(a + l1b_ref[...])
    a = jnp.dot(a, l2WT_ref[...], preferred_element_type=jnp.float32)
    a = jax.nn.relu(a + l2b_ref[...])
    a = jnp.dot(a, l3WT_ref[...], preferred_element_type=jnp.float32)
    out_ref[...] = jax.nn.sigmoid(a + l3b_ref[...])


def _mlp(agg, h, WlT, bl, WrT, l1WT, l1b, l2WT, l2b, l3WT, l3b):
    return pl.pallas_call(
        _mlp_body,
        grid=(N // ROW_BLK,),
        in_specs=[
            pl.BlockSpec((ROW_BLK, D), lambda i: (i, 0)),
            pl.BlockSpec((ROW_BLK, D), lambda i: (i, 0)),
            pl.BlockSpec((D, D), lambda i: (0, 0)),
            pl.BlockSpec((1, D), lambda i: (0, 0)),
            pl.BlockSpec((D, D), lambda i: (0, 0)),
            pl.BlockSpec((D, D), lambda i: (0, 0)),
            pl.BlockSpec((1, D), lambda i: (0, 0)),
            pl.BlockSpec((D, D), lambda i: (0, 0)),
            pl.BlockSpec((1, D), lambda i: (0, 0)),
            pl.BlockSpec((D, D), lambda i: (0, 0)),
            pl.BlockSpec((1, D), lambda i: (0, 0)),
        ],
        out_specs=pl.BlockSpec((ROW_BLK, D), lambda i: (i, 0)),
        out_shape=jax.ShapeDtypeStruct((N, D), jnp.float32),
    )(agg, h, WlT, bl, WrT, l1WT, l1b, l2WT, l2b, l3WT, l3b)


def kernel(x, edge_index, batch, W1l, b1l, W1r, W2l, b2l, W2r, W3l, b3l, W3r,
           l1W, l1b, l2W, l2b, l3W, l3b):
    src = edge_index[0]
    dst = edge_index[1]

    comp_src, comp_dst, counts, sorted_src, rowptr = _partition_edges(src, dst)


    h = x
    for Wl, bl, Wr in ((W1l, b1l, W1r), (W2l, b2l, W2r)):
        agg = _segmax_sc(h, comp_src, comp_dst, counts, sorted_src, rowptr)[:N]
        h = _sage_dense(agg, h, Wl.T, bl[None, :], Wr.T)
    agg = _segmax_sc(h, comp_src, comp_dst, counts, sorted_src, rowptr)[:N]

    l2WT = jnp.pad(l2W.T, ((0, 0), (0, 64)))
    l2b_p = jnp.pad(l2b, (0, 64))[None, :]
    l3WT = jnp.pad(l3W.T, ((0, 64), (0, 122)))
    l3b_p = jnp.pad(l3b, (0, 122))[None, :]
    out = _mlp(agg, h, W3l.T, b3l[None, :], W3r.T,
               l1W.T, l1b[None, :], l2WT, l2b_p, l3WT, l3b_p)
    return out[:, :6]
